# Initial kernel scaffold; baseline (speedup 1.0000x reference)
import jax
import jax.numpy as jnp
from jax.experimental import pallas as pl


def kernel(x, edge_index, edge_attr, We1, be1, W11, b11, W12, b12, We2, be2, W21, b21, W22, b22, g1, bb1, g2, bb2, Wl, bl):
    raise NotImplementedError("write your pallas kernel here")


# speedup vs baseline: 1.8276x; 1.8276x over previous
import functools

import jax
import jax.numpy as jnp
from jax import lax
from jax.experimental import pallas as pl
from jax.experimental.pallas import tpu as pltpu
from jax.experimental.pallas import tpu_sc as plsc

_N = 10000
_E = 320000
_F = 128
_FH = 64
_NC, _NS = 2, 16
_EW = _E // _NS
_C = 80
_NCHUNK = _EW // _C
_ZR = 208
_NROWS = 624
_TAIL = _N - _NS * _NROWS


def _sc_agg(ta, tb, ea, eb, src_rs, dst_rs):
    mesh = plsc.VectorSubcoreMesh(core_axis_name="c", subcore_axis_name="s")

    @functools.partial(
        pl.kernel,
        out_type=jax.ShapeDtypeStruct((2 * _N, _FH), jnp.float32),
        mesh=mesh,
        scratch_types=[
            pltpu.VMEM((_NCHUNK, _C), jnp.int32),
            pltpu.VMEM((_NCHUNK, _C), jnp.int32),
            pltpu.VMEM((_C, _FH), jnp.float32),
            pltpu.VMEM((_C, _FH), jnp.float32),
            pltpu.VMEM((_ZR, _FH), jnp.float32),
            pltpu.VMEM_SHARED((_N, _FH), jnp.float32),
            pltpu.SemaphoreType.DMA,
            pltpu.SemaphoreType.DMA,
        ],
        compiler_params=pltpu.CompilerParams(use_tc_tiling_on_sc=False),
    )
    def k(ta_hbm, tb_hbm, ea_hbm, eb_hbm, src_hbm, dst_hbm, out_hbm,
          sidx, didx, xg, ev, zbuf, acc, gsem, esem):
        cid = lax.axis_index("c")
        sid = lax.axis_index("s")

        def zrow(r, c):
            for j in range(_FH // 16):
                zbuf[r, pl.ds(j * 16, 16)] = jnp.zeros((16,), jnp.float32)
            return c
        lax.fori_loop(0, _ZR, zrow, 0)
        row0 = sid * _NROWS
        for kk in range(_NROWS // _ZR):
            pltpu.sync_copy(zbuf, acc.at[pl.ds(row0 + kk * _ZR, _ZR)])

        @pl.when(sid == 0)
        def _():
            pltpu.sync_copy(zbuf.at[pl.ds(0, _TAIL)],
                            acc.at[pl.ds(_NS * _NROWS, _TAIL)])

        pltpu.sync_copy(src_hbm.at[sid], sidx)
        pltpu.sync_copy(dst_hbm.at[sid], didx)
        plsc.subcore_barrier()

        ebase = sid * _EW

        def run(t_hbm, e_hbm):
            def chunk(i, c):
                g = pltpu.async_copy(t_hbm.at[sidx.at[i]], xg, gsem)
                el = pltpu.async_copy(
                    e_hbm.at[pl.ds(ebase + i * _C, _C)], ev, esem)
                g.wait()
                el.wait()

                def crow(r, c2):
                    for j in range(_FH // 16):
                        sl = pl.ds(j * 16, 16)
                        xg[r, sl] = jnp.maximum(xg[r, sl] + ev[r, sl], 0.0)
                    return c2
                lax.fori_loop(0, _C, crow, 0)
                pltpu.sync_copy(xg, acc.at[didx.at[i]], add=True)
                return c
            lax.fori_loop(0, _NCHUNK, chunk, 0)

        @pl.when(cid == 0)
        def _():
            run(ta_hbm, ea_hbm)

        @pl.when(cid == 1)
        def _():
            run(tb_hbm, eb_hbm)

        plsc.subcore_barrier()

        pltpu.sync_copy(acc.at[pl.ds(row0, _NROWS)],
                        out_hbm.at[pl.ds(cid * _N + row0, _NROWS)])

        @pl.when(sid == 0)
        def _():
            pltpu.sync_copy(acc.at[pl.ds(_NS * _NROWS, _TAIL)],
                            out_hbm.at[pl.ds(cid * _N + _NS * _NROWS, _TAIL)])

    return k(ta, tb, ea, eb, src_rs, dst_rs)


def _agg128(t128, ea, eb, src_rs, dst_rs):
    out = _sc_agg(t128[:, :_FH], t128[:, _FH:], ea, eb, src_rs, dst_rs)
    return jnp.concatenate([out[:_N], out[_N:]], axis=1)


_BE = 1280
_BN = 400


def _edge_body(ea, w1, b1, w2, b2, o1a, o1b, o2a, o2b, o2c, o2d):
    a = ea[...]
    e1 = jnp.dot(a, w1[...], preferred_element_type=jnp.float32) + b1[...]
    o1a[...] = e1[:, :_FH]
    o1b[...] = e1[:, _FH:]
    e2 = jnp.dot(a, w2[...], preferred_element_type=jnp.float32) + b2[...]
    o2a[...] = e2[:, 0 * _FH:1 * _FH]
    o2b[...] = e2[:, 1 * _FH:2 * _FH]
    o2c[...] = e2[:, 2 * _FH:3 * _FH]
    o2d[...] = e2[:, 3 * _FH:4 * _FH]


def _edge_transform(edge_attr, We1, be1, We2, be2):
    ed = edge_attr.shape[1]
    grid = _E // _BE
    hblk = pl.BlockSpec((_BE, _FH), lambda i: (i, 0))
    return pl.pallas_call(
        _edge_body,
        grid=(grid,),
        in_specs=[
            pl.BlockSpec((_BE, ed), lambda i: (i, 0)),
            pl.BlockSpec((ed, _F), lambda i: (0, 0)),
            pl.BlockSpec((1, _F), lambda i: (0, 0)),
            pl.BlockSpec((ed, 2 * _F), lambda i: (0, 0)),
            pl.BlockSpec((1, 2 * _F), lambda i: (0, 0)),
        ],
        out_specs=[hblk] * 6,
        out_shape=[jax.ShapeDtypeStruct((_E, _FH), jnp.float32)] * 6,
    )(edge_attr, We1, be1, We2, be2)


def _mlp1_body(x, p, w11, b11, w12, b12, s1, t1, oa, ob):
    h = x[...] + p[...]
    a = jnp.maximum(
        jnp.dot(h, w11[...], preferred_element_type=jnp.float32) + b11[...], 0.0)
    z = jnp.dot(a, w12[...], preferred_element_type=jnp.float32) + b12[...]
    hb = jnp.maximum(z, 0.0) * s1[...] + t1[...]
    oa[...] = hb[:, :_F]
    ob[...] = hb[:, _F:]


def _mlp1(x, p, W11, b11, W12, b12, s1, t1):
    grid = _N // _BN
    return pl.pallas_call(
        _mlp1_body,
        grid=(grid,),
        in_specs=[
            pl.BlockSpec((_BN, _F), lambda i: (i, 0)),
            pl.BlockSpec((_BN, _F), lambda i: (i, 0)),
            pl.BlockSpec((_F, 2 * _F), lambda i: (0, 0)),
            pl.BlockSpec((1, 2 * _F), lambda i: (0, 0)),
            pl.BlockSpec((2 * _F, 2 * _F), lambda i: (0, 0)),
            pl.BlockSpec((1, 2 * _F), lambda i: (0, 0)),
            pl.BlockSpec((1, 2 * _F), lambda i: (0, 0)),
            pl.BlockSpec((1, 2 * _F), lambda i: (0, 0)),
        ],
        out_specs=[
            pl.BlockSpec((_BN, _F), lambda i: (i, 0)),
            pl.BlockSpec((_BN, _F), lambda i: (i, 0)),
        ],
        out_shape=[
            jax.ShapeDtypeStruct((_N, _F), jnp.float32),
            jax.ShapeDtypeStruct((_N, _F), jnp.float32),
        ],
    )(x, p, W11, b11, W12, b12, s1, t1)


def _mlp2_body(ha, hb, qa, qb, w21a, w21b, b21, w22, b22, s2, t2, wl, bl, o):
    h2a = ha[...] + qa[...]
    h2b = hb[...] + qb[...]
    a = jnp.maximum(
        jnp.dot(h2a, w21a[...], preferred_element_type=jnp.float32)
        + jnp.dot(h2b, w21b[...], preferred_element_type=jnp.float32)
        + b21[...], 0.0)
    z = jnp.dot(a, w22[...], preferred_element_type=jnp.float32) + b22[...]
    hc = jnp.maximum(z, 0.0) * s2[...] + t2[...]
    o[...] = jnp.dot(hc, wl[...], preferred_element_type=jnp.float32) + bl[...]


def _mlp2(ha, hb, qa, qb, W21a, W21b, b21, W22, b22, s2, t2, Wl, bl):
    grid = _N // _BN
    blk = pl.BlockSpec((_BN, _F), lambda i: (i, 0))
    full = lambda shape: pl.BlockSpec(shape, lambda i: (0, 0))
    return pl.pallas_call(
        _mlp2_body,
        grid=(grid,),
        in_specs=[
            blk, blk, blk, blk,
            full((_F, 2 * _F)),
            full((_F, 2 * _F)),
            full((1, 2 * _F)),
            full((2 * _F, 2 * _F)),
            full((1, 2 * _F)),
            full((1, 2 * _F)),
            full((1, 2 * _F)),
            full((2 * _F, _F)),
            full((1, _F)),
        ],
        out_specs=blk,
        out_shape=jax.ShapeDtypeStruct((_N, _F), jnp.float32),
    )(ha, hb, qa, qb, W21a, W21b, b21, W22, b22, s2, t2, Wl, bl)


def kernel(x, edge_index, edge_attr, We1, be1, W11, b11, W12, b12,
           We2, be2, W21, b21, W22, b22, g1, bb1, g2, bb2, Wl, bl):
    src = edge_index[0].reshape(_NS, _NCHUNK, _C)
    dst = edge_index[1].reshape(_NS, _NCHUNK, _C)
    inv = 1.0 / jnp.sqrt(jnp.float32(1.0 + 1e-5))
    s1 = (g1 * inv).reshape(1, -1)
    s2 = (g2 * inv).reshape(1, -1)

    e1a, e1b, e2a, e2b, e2c, e2d = _edge_transform(
        edge_attr, We1, be1.reshape(1, -1), We2, be2.reshape(1, -1))

    p = _agg128(x, e1a, e1b, src, dst)
    h1a, h1b = _mlp1(x, p, W11, b11.reshape(1, -1), W12, b12.reshape(1, -1),
                     s1, bb1.reshape(1, -1))

    qa = _agg128(h1a, e2a, e2b, src, dst)
    qb = _agg128(h1b, e2c, e2d, src, dst)

    return _mlp2(h1a, h1b, qa, qb, W21[:_F], W21[_F:], b21.reshape(1, -1),
                 W22, b22.reshape(1, -1), s2, bb2.reshape(1, -1),
                 Wl, bl.reshape(1, -1))


# speedup vs baseline: 2.2996x; 1.2582x over previous
import functools

import jax
import jax.numpy as jnp
from jax import lax
from jax.experimental import pallas as pl
from jax.experimental.pallas import tpu as pltpu
from jax.experimental.pallas import tpu_sc as plsc

_N = 10000
_E = 320000
_F = 128
_FH = 64
_NC, _NS = 2, 16
_EW = _E // _NS
_C = 80
_NCHUNK = _EW // _C
_ZR = 208
_NROWS = 624
_TAIL = _N - _NS * _NROWS


def _sc_agg(ta, tb, e, src_rs, dst_rs):
    mesh = plsc.VectorSubcoreMesh(core_axis_name="c", subcore_axis_name="s")

    @functools.partial(
        pl.kernel,
        out_type=jax.ShapeDtypeStruct((2 * _N, _FH), jnp.float32),
        mesh=mesh,
        scratch_types=[
            pltpu.VMEM((_NCHUNK, _C), jnp.int32),
            pltpu.VMEM((_NCHUNK, _C), jnp.int32),
            pltpu.VMEM((_C, _FH), jnp.float32),
            pltpu.VMEM((_C, _FH), jnp.float32),
            pltpu.VMEM((_ZR, _FH), jnp.float32),
            pltpu.VMEM_SHARED((_N, _FH), jnp.float32),
            pltpu.SemaphoreType.DMA,
            pltpu.SemaphoreType.DMA,
        ],
        compiler_params=pltpu.CompilerParams(use_tc_tiling_on_sc=False),
    )
    def k(ta_hbm, tb_hbm, e_hbm, src_hbm, dst_hbm, out_hbm,
          sidx, didx, xg, ev, zbuf, acc, gsem, esem):
        cid = lax.axis_index("c")
        sid = lax.axis_index("s")
        col0 = cid * _FH

        def zrow(r, c):
            for j in range(_FH // 16):
                zbuf[r, pl.ds(j * 16, 16)] = jnp.zeros((16,), jnp.float32)
            return c
        lax.fori_loop(0, _ZR, zrow, 0)
        row0 = sid * _NROWS
        for kk in range(_NROWS // _ZR):
            pltpu.sync_copy(zbuf, acc.at[pl.ds(row0 + kk * _ZR, _ZR)])

        @pl.when(sid == 0)
        def _():
            pltpu.sync_copy(zbuf.at[pl.ds(0, _TAIL)],
                            acc.at[pl.ds(_NS * _NROWS, _TAIL)])

        pltpu.sync_copy(src_hbm.at[sid], sidx)
        pltpu.sync_copy(dst_hbm.at[sid], didx)
        plsc.subcore_barrier()

        ebase = sid * _EW

        def run(t_hbm):
            def chunk(i, c):
                g = pltpu.async_copy(t_hbm.at[sidx.at[i]], xg, gsem)
                el = pltpu.async_copy(
                    e_hbm.at[pl.ds(ebase + i * _C, _C), pl.ds(col0, _FH)],
                    ev, esem)
                g.wait()
                el.wait()

                def crow(r, c2):
                    for j in range(_FH // 16):
                        sl = pl.ds(j * 16, 16)
                        xg[r, sl] = jnp.maximum(xg[r, sl] + ev[r, sl], 0.0)
                    return c2
                lax.fori_loop(0, _C, crow, 0)
                pltpu.sync_copy(xg, acc.at[didx.at[i]], add=True)
                return c
            lax.fori_loop(0, _NCHUNK, chunk, 0)

        @pl.when(cid == 0)
        def _():
            run(ta_hbm)

        @pl.when(cid == 1)
        def _():
            run(tb_hbm)

        plsc.subcore_barrier()

        pltpu.sync_copy(acc.at[pl.ds(row0, _NROWS)],
                        out_hbm.at[pl.ds(cid * _N + row0, _NROWS)])

        @pl.when(sid == 0)
        def _():
            pltpu.sync_copy(acc.at[pl.ds(_NS * _NROWS, _TAIL)],
                            out_hbm.at[pl.ds(cid * _N + _NS * _NROWS, _TAIL)])

    return k(ta, tb, e, src_rs, dst_rs)


def _agg128(t128, e, src_rs, dst_rs):
    out = _sc_agg(t128[:, :_FH], t128[:, _FH:], e, src_rs, dst_rs)
    return jnp.concatenate([out[:_N], out[_N:]], axis=1)


_BE = 1280
_BN = 400


def _edge_body(ea, w1, b1, w2, b2, o1, o2a, o2b):
    a = ea[...]
    o1[...] = jnp.dot(a, w1[...], preferred_element_type=jnp.float32) + b1[...]
    e2 = jnp.dot(a, w2[...], preferred_element_type=jnp.float32) + b2[...]
    o2a[...] = e2[:, :_F]
    o2b[...] = e2[:, _F:]


def _edge_transform(edge_attr, We1, be1, We2, be2):
    ed = edge_attr.shape[1]
    grid = _E // _BE
    fblk = pl.BlockSpec((_BE, _F), lambda i: (i, 0))
    return pl.pallas_call(
        _edge_body,
        grid=(grid,),
        in_specs=[
            pl.BlockSpec((_BE, ed), lambda i: (i, 0)),
            pl.BlockSpec((ed, _F), lambda i: (0, 0)),
            pl.BlockSpec((1, _F), lambda i: (0, 0)),
            pl.BlockSpec((ed, 2 * _F), lambda i: (0, 0)),
            pl.BlockSpec((1, 2 * _F), lambda i: (0, 0)),
        ],
        out_specs=[fblk] * 3,
        out_shape=[jax.ShapeDtypeStruct((_E, _F), jnp.float32)] * 3,
    )(edge_attr, We1, be1, We2, be2)


def _mlp1_body(x, p, w11, b11, w12, b12, s1, t1, oa, ob):
    h = x[...] + p[...]
    a = jnp.maximum(
        jnp.dot(h, w11[...], preferred_element_type=jnp.float32) + b11[...], 0.0)
    z = jnp.dot(a, w12[...], preferred_element_type=jnp.float32) + b12[...]
    hb = jnp.maximum(z, 0.0) * s1[...] + t1[...]
    oa[...] = hb[:, :_F]
    ob[...] = hb[:, _F:]


def _mlp1(x, p, W11, b11, W12, b12, s1, t1):
    grid = _N // _BN
    return pl.pallas_call(
        _mlp1_body,
        grid=(grid,),
        in_specs=[
            pl.BlockSpec((_BN, _F), lambda i: (i, 0)),
            pl.BlockSpec((_BN, _F), lambda i: (i, 0)),
            pl.BlockSpec((_F, 2 * _F), lambda i: (0, 0)),
            pl.BlockSpec((1, 2 * _F), lambda i: (0, 0)),
            pl.BlockSpec((2 * _F, 2 * _F), lambda i: (0, 0)),
            pl.BlockSpec((1, 2 * _F), lambda i: (0, 0)),
            pl.BlockSpec((1, 2 * _F), lambda i: (0, 0)),
            pl.BlockSpec((1, 2 * _F), lambda i: (0, 0)),
        ],
        out_specs=[
            pl.BlockSpec((_BN, _F), lambda i: (i, 0)),
            pl.BlockSpec((_BN, _F), lambda i: (i, 0)),
        ],
        out_shape=[
            jax.ShapeDtypeStruct((_N, _F), jnp.float32),
            jax.ShapeDtypeStruct((_N, _F), jnp.float32),
        ],
    )(x, p, W11, b11, W12, b12, s1, t1)


def _mlp2_body(ha, hb, qa, qb, w21a, w21b, b21, w22, b22, s2, t2, wl, bl, o):
    h2a = ha[...] + qa[...]
    h2b = hb[...] + qb[...]
    a = jnp.maximum(
        jnp.dot(h2a, w21a[...], preferred_element_type=jnp.float32)
        + jnp.dot(h2b, w21b[...], preferred_element_type=jnp.float32)
        + b21[...], 0.0)
    z = jnp.dot(a, w22[...], preferred_element_type=jnp.float32) + b22[...]
    hc = jnp.maximum(z, 0.0) * s2[...] + t2[...]
    o[...] = jnp.dot(hc, wl[...], preferred_element_type=jnp.float32) + bl[...]


def _mlp2(ha, hb, qa, qb, W21a, W21b, b21, W22, b22, s2, t2, Wl, bl):
    grid = _N // _BN
    blk = pl.BlockSpec((_BN, _F), lambda i: (i, 0))
    full = lambda shape: pl.BlockSpec(shape, lambda i: (0, 0))
    return pl.pallas_call(
        _mlp2_body,
        grid=(grid,),
        in_specs=[
            blk, blk, blk, blk,
            full((_F, 2 * _F)),
            full((_F, 2 * _F)),
            full((1, 2 * _F)),
            full((2 * _F, 2 * _F)),
            full((1, 2 * _F)),
            full((1, 2 * _F)),
            full((1, 2 * _F)),
            full((2 * _F, _F)),
            full((1, _F)),
        ],
        out_specs=blk,
        out_shape=jax.ShapeDtypeStruct((_N, _F), jnp.float32),
    )(ha, hb, qa, qb, W21a, W21b, b21, W22, b22, s2, t2, Wl, bl)


def kernel(x, edge_index, edge_attr, We1, be1, W11, b11, W12, b12,
           We2, be2, W21, b21, W22, b22, g1, bb1, g2, bb2, Wl, bl):
    src = edge_index[0].reshape(_NS, _NCHUNK, _C)
    dst = edge_index[1].reshape(_NS, _NCHUNK, _C)
    inv = 1.0 / jnp.sqrt(jnp.float32(1.0 + 1e-5))
    s1 = (g1 * inv).reshape(1, -1)
    s2 = (g2 * inv).reshape(1, -1)

    e1, e2a, e2b = _edge_transform(
        edge_attr, We1, be1.reshape(1, -1), We2, be2.reshape(1, -1))

    p = _agg128(x, e1, src, dst)
    h1a, h1b = _mlp1(x, p, W11, b11.reshape(1, -1), W12, b12.reshape(1, -1),
                     s1, bb1.reshape(1, -1))

    qa = _agg128(h1a, e2a, src, dst)
    qb = _agg128(h1b, e2b, src, dst)

    return _mlp2(h1a, h1b, qa, qb, W21[:_F], W21[_F:], b21.reshape(1, -1),
                 W22, b22.reshape(1, -1), s2, bb2.reshape(1, -1),
                 Wl, bl.reshape(1, -1))


# speedup vs baseline: 3.6772x; 1.5991x over previous
import functools

import jax
import jax.numpy as jnp
from jax import lax
from jax.experimental import pallas as pl
from jax.experimental.pallas import tpu as pltpu
from jax.experimental.pallas import tpu_sc as plsc

_N = 10000
_E = 320000
_F = 128
_FH = 64
_NC, _NS = 2, 16
_EW = _E // _NS
_C = 80
_NCHUNK = _EW // _C
_ZR = 208
_NROWS = 624
_TAIL = _N - _NS * _NROWS


def _sc_agg(ta, tb, e, src_rs, dst_rs):
    mesh = plsc.VectorSubcoreMesh(core_axis_name="c", subcore_axis_name="s")

    @functools.partial(
        pl.kernel,
        out_type=jax.ShapeDtypeStruct((2 * _N, _FH), jnp.float32),
        mesh=mesh,
        scratch_types=[
            pltpu.VMEM((_NCHUNK, _C), jnp.int32),
            pltpu.VMEM((_NCHUNK, _C), jnp.int32),
            pltpu.VMEM((_C, _FH), jnp.float32),
            pltpu.VMEM((_C, _FH), jnp.float32),
            pltpu.VMEM((_C, _FH), jnp.float32),
            pltpu.VMEM((_C, _FH), jnp.float32),
            pltpu.VMEM((_C, _FH), jnp.float32),
            pltpu.VMEM((_ZR, _FH), jnp.float32),
            pltpu.VMEM_SHARED((_N, _FH), jnp.float32),
            pltpu.SemaphoreType.DMA,
            pltpu.SemaphoreType.DMA,
            pltpu.SemaphoreType.DMA,
            pltpu.SemaphoreType.DMA,
        ],
        compiler_params=pltpu.CompilerParams(use_tc_tiling_on_sc=False),
    )
    def k(ta_hbm, tb_hbm, e_hbm, src_hbm, dst_hbm, out_hbm,
          sidx, didx, xg0, xg1, ev0, ev1, msg, zbuf, acc,
          gsem0, gsem1, esem0, esem1):
        cid = lax.axis_index("c")
        sid = lax.axis_index("s")
        col0 = cid * _FH

        def zrow(r, c):
            for j in range(_FH // 16):
                zbuf[r, pl.ds(j * 16, 16)] = jnp.zeros((16,), jnp.float32)
            return c
        lax.fori_loop(0, _ZR, zrow, 0)
        row0 = sid * _NROWS
        for kk in range(_NROWS // _ZR):
            pltpu.sync_copy(zbuf, acc.at[pl.ds(row0 + kk * _ZR, _ZR)])

        @pl.when(sid == 0)
        def _():
            pltpu.sync_copy(zbuf.at[pl.ds(0, _TAIL)],
                            acc.at[pl.ds(_NS * _NROWS, _TAIL)])

        pltpu.sync_copy(src_hbm.at[sid], sidx)
        pltpu.sync_copy(dst_hbm.at[sid], didx)
        plsc.subcore_barrier()

        ebase = sid * _EW
        bufs = ((xg0, ev0, gsem0, esem0), (xg1, ev1, gsem1, esem1))

        def run(t_hbm):
            def eslc(i):
                return e_hbm.at[pl.ds(ebase + i * _C, _C), pl.ds(col0, _FH)]

            def start(i, xg, ev, gsem, esem):
                pltpu.async_copy(t_hbm.at[sidx.at[i]], xg, gsem)
                pltpu.async_copy(eslc(i), ev, esem)

            start(0, *bufs[0])
            start(1, *bufs[1])

            def pair(kk, c):
                i0 = kk * 2
                for p, (xg, ev, gsem, esem) in enumerate(bufs):
                    i = i0 + p
                    pltpu.make_async_copy(t_hbm.at[sidx.at[i]], xg, gsem).wait()
                    pltpu.make_async_copy(eslc(i), ev, esem).wait()

                    def crow(r, c2, xg=xg, ev=ev):
                        for j in range(_FH // 16):
                            sl = pl.ds(j * 16, 16)
                            msg[r, sl] = jnp.maximum(xg[r, sl] + ev[r, sl], 0.0)
                        return c2
                    lax.fori_loop(0, _C, crow, 0)

                    @pl.when(i + 2 < _NCHUNK)
                    def _(i=i, xg=xg, ev=ev, gsem=gsem, esem=esem):
                        start(i + 2, xg, ev, gsem, esem)

                    pltpu.sync_copy(msg, acc.at[didx.at[i]], add=True)
                return c
            lax.fori_loop(0, _NCHUNK // 2, pair, 0)

        @pl.when(cid == 0)
        def _():
            run(ta_hbm)

        @pl.when(cid == 1)
        def _():
            run(tb_hbm)

        plsc.subcore_barrier()

        pltpu.sync_copy(acc.at[pl.ds(row0, _NROWS)],
                        out_hbm.at[pl.ds(cid * _N + row0, _NROWS)])

        @pl.when(sid == 0)
        def _():
            pltpu.sync_copy(acc.at[pl.ds(_NS * _NROWS, _TAIL)],
                            out_hbm.at[pl.ds(cid * _N + _NS * _NROWS, _TAIL)])

    return k(ta, tb, e, src_rs, dst_rs)


def _agg128(t128, e, src_rs, dst_rs):
    out = _sc_agg(t128[:, :_FH], t128[:, _FH:], e, src_rs, dst_rs)
    return jnp.concatenate([out[:_N], out[_N:]], axis=1)


_BE = 1280
_BN = 400


def _edge_body(ea, w1, b1, w2, b2, o1, o2a, o2b):
    a = ea[...]
    o1[...] = jnp.dot(a, w1[...], preferred_element_type=jnp.float32) + b1[...]
    e2 = jnp.dot(a, w2[...], preferred_element_type=jnp.float32) + b2[...]
    o2a[...] = e2[:, :_F]
    o2b[...] = e2[:, _F:]


def _edge_transform(edge_attr, We1, be1, We2, be2):
    ed = edge_attr.shape[1]
    grid = _E // _BE
    fblk = pl.BlockSpec((_BE, _F), lambda i: (i, 0))
    return pl.pallas_call(
        _edge_body,
        grid=(grid,),
        in_specs=[
            pl.BlockSpec((_BE, ed), lambda i: (i, 0)),
            pl.BlockSpec((ed, _F), lambda i: (0, 0)),
            pl.BlockSpec((1, _F), lambda i: (0, 0)),
            pl.BlockSpec((ed, 2 * _F), lambda i: (0, 0)),
            pl.BlockSpec((1, 2 * _F), lambda i: (0, 0)),
        ],
        out_specs=[fblk] * 3,
        out_shape=[jax.ShapeDtypeStruct((_E, _F), jnp.float32)] * 3,
    )(edge_attr, We1, be1, We2, be2)


def _mlp1_body(x, p, w11, b11, w12, b12, s1, t1, oa, ob):
    h = x[...] + p[...]
    a = jnp.maximum(
        jnp.dot(h, w11[...], preferred_element_type=jnp.float32) + b11[...], 0.0)
    z = jnp.dot(a, w12[...], preferred_element_type=jnp.float32) + b12[...]
    hb = jnp.maximum(z, 0.0) * s1[...] + t1[...]
    oa[...] = hb[:, :_F]
    ob[...] = hb[:, _F:]


def _mlp1(x, p, W11, b11, W12, b12, s1, t1):
    grid = _N // _BN
    return pl.pallas_call(
        _mlp1_body,
        grid=(grid,),
        in_specs=[
            pl.BlockSpec((_BN, _F), lambda i: (i, 0)),
            pl.BlockSpec((_BN, _F), lambda i: (i, 0)),
            pl.BlockSpec((_F, 2 * _F), lambda i: (0, 0)),
            pl.BlockSpec((1, 2 * _F), lambda i: (0, 0)),
            pl.BlockSpec((2 * _F, 2 * _F), lambda i: (0, 0)),
            pl.BlockSpec((1, 2 * _F), lambda i: (0, 0)),
            pl.BlockSpec((1, 2 * _F), lambda i: (0, 0)),
            pl.BlockSpec((1, 2 * _F), lambda i: (0, 0)),
        ],
        out_specs=[
            pl.BlockSpec((_BN, _F), lambda i: (i, 0)),
            pl.BlockSpec((_BN, _F), lambda i: (i, 0)),
        ],
        out_shape=[
            jax.ShapeDtypeStruct((_N, _F), jnp.float32),
            jax.ShapeDtypeStruct((_N, _F), jnp.float32),
        ],
    )(x, p, W11, b11, W12, b12, s1, t1)


def _mlp2_body(ha, hb, qa, qb, w21a, w21b, b21, w22, b22, s2, t2, wl, bl, o):
    h2a = ha[...] + qa[...]
    h2b = hb[...] + qb[...]
    a = jnp.maximum(
        jnp.dot(h2a, w21a[...], preferred_element_type=jnp.float32)
        + jnp.dot(h2b, w21b[...], preferred_element_type=jnp.float32)
        + b21[...], 0.0)
    z = jnp.dot(a, w22[...], preferred_element_type=jnp.float32) + b22[...]
    hc = jnp.maximum(z, 0.0) * s2[...] + t2[...]
    o[...] = jnp.dot(hc, wl[...], preferred_element_type=jnp.float32) + bl[...]


def _mlp2(ha, hb, qa, qb, W21a, W21b, b21, W22, b22, s2, t2, Wl, bl):
    grid = _N // _BN
    blk = pl.BlockSpec((_BN, _F), lambda i: (i, 0))
    full = lambda shape: pl.BlockSpec(shape, lambda i: (0, 0))
    return pl.pallas_call(
        _mlp2_body,
        grid=(grid,),
        in_specs=[
            blk, blk, blk, blk,
            full((_F, 2 * _F)),
            full((_F, 2 * _F)),
            full((1, 2 * _F)),
            full((2 * _F, 2 * _F)),
            full((1, 2 * _F)),
            full((1, 2 * _F)),
            full((1, 2 * _F)),
            full((2 * _F, _F)),
            full((1, _F)),
        ],
        out_specs=blk,
        out_shape=jax.ShapeDtypeStruct((_N, _F), jnp.float32),
    )(ha, hb, qa, qb, W21a, W21b, b21, W22, b22, s2, t2, Wl, bl)


def kernel(x, edge_index, edge_attr, We1, be1, W11, b11, W12, b12,
           We2, be2, W21, b21, W22, b22, g1, bb1, g2, bb2, Wl, bl):
    src = edge_index[0].reshape(_NS, _NCHUNK, _C)
    dst = edge_index[1].reshape(_NS, _NCHUNK, _C)
    inv = 1.0 / jnp.sqrt(jnp.float32(1.0 + 1e-5))
    s1 = (g1 * inv).reshape(1, -1)
    s2 = (g2 * inv).reshape(1, -1)

    e1, e2a, e2b = _edge_transform(
        edge_attr, We1, be1.reshape(1, -1), We2, be2.reshape(1, -1))

    p = _agg128(x, e1, src, dst)
    h1a, h1b = _mlp1(x, p, W11, b11.reshape(1, -1), W12, b12.reshape(1, -1),
                     s1, bb1.reshape(1, -1))

    qa = _agg128(h1a, e2a, src, dst)
    qb = _agg128(h1b, e2b, src, dst)

    return _mlp2(h1a, h1b, qa, qb, W21[:_F], W21[_F:], b21.reshape(1, -1),
                 W22, b22.reshape(1, -1), s2, bb2.reshape(1, -1),
                 Wl, bl.reshape(1, -1))
